# baseline (speedup 1.0000x reference)
import functools

import jax
import jax.numpy as jnp
from jax import lax
from jax.experimental import pallas as pl
from jax.experimental.pallas import tpu as pltpu
from jax.experimental.pallas import tpu_sc as plsc

B = 4096
MARGIN = 0.8
K = 3
NEG = -3.0e38

S_TC = 2560


L = 16
NC, NS = 2, 16
NW = NC * NS
RPW = (B - S_TC) // NW
CH = 8
NCHUNK = RPW // CH
NVREG = B // L


def _tec_body(in_hbm, out_hbm, buf, stage, sem0, sem1):
    cid = lax.axis_index("c")
    sid = lax.axis_index("s")
    wid = sid * NC + cid
    base = S_TC + wid * RPW
    lane = lax.broadcasted_iota(jnp.int32, (L,), 0)
    neg = jnp.full((L,), NEG, jnp.float32)

    sems = (sem0, sem1)

    def start_chunk(k, slot):
        pltpu.async_copy(
            in_hbm.at[pl.ds(base + k * CH, CH)], buf.at[slot], sems[slot])

    def wait_chunk(k, slot):
        pltpu.make_async_copy(
            in_hbm.at[pl.ds(base + k * CH, CH)], buf.at[slot],
            sems[slot]).wait()

    def merge(t, x):
        t1, t2, t3 = t
        n1 = jnp.maximum(t1, x)
        lo = jnp.minimum(t1, x)
        n2 = jnp.maximum(t2, lo)
        lo2 = jnp.minimum(t2, lo)
        n3 = jnp.maximum(t3, lo2)
        return (n1, n2, n3)

    U = 16
    A = 4

    def process_row(i_global, slot, r_local, acc):
        c = i_global // L
        ldiag = i_global % L
        v = buf[slot, r_local, pl.ds(c * L, L)]
        dmask = lane == ldiag
        d = jnp.max(jnp.where(dmask, v, neg))
        buf[slot, r_local, pl.ds(c * L, L)] = jnp.where(dmask, neg, v)

        def scan_body(s, carry):
            ts = list(carry)
            for u in range(U):
                x = buf[slot, r_local, pl.ds((s * U + u) * L, L)]
                ts[u % A] = merge(ts[u % A], x)
            return tuple(ts)

        init = tuple((neg, neg, neg) for _ in range(A))
        ts = lax.fori_loop(0, NVREG // U, scan_body, init)
        t = ts[0]
        for a in range(1, A):
            o1, o2, o3 = ts[a]
            t = merge(merge(merge(t, o1), o2), o3)
        t1, t2, t3 = t

        for _ in range(K):
            m = jnp.max(t1)
            sel = lane == plsc.all_reduce_ffs(t1 == m)
            acc = acc + jnp.maximum(m - d + MARGIN, 0.0)
            t1 = jnp.where(sel, t2, t1)
            t2 = jnp.where(sel, t3, t2)
            t3 = jnp.where(sel, NEG, t3)
        return acc

    def chunk_pair(p, acc):
        start_chunk(2 * p + 1, 1)
        wait_chunk(2 * p, 0)

        def rows0(r, a):
            return process_row(base + (2 * p) * CH + r, 0, r, a)

        acc = lax.fori_loop(0, CH, rows0, acc)

        @pl.when(p + 1 < NCHUNK // 2)
        def _():
            start_chunk(2 * p + 2, 0)

        wait_chunk(2 * p + 1, 1)

        def rows1(r, a):
            return process_row(base + (2 * p + 1) * CH + r, 1, r, a)

        return lax.fori_loop(0, CH, rows1, acc)

    start_chunk(0, 0)
    acc = lax.fori_loop(0, NCHUNK // 2, chunk_pair, jnp.float32(0.0))

    stage[...] = jnp.where(lane == 0, acc, 0.0)
    pltpu.sync_copy(stage, out_hbm.at[wid])


_mesh = plsc.VectorSubcoreMesh(core_axis_name="c", subcore_axis_name="s")

_sc_loss = pl.kernel(
    _tec_body,
    out_type=jax.ShapeDtypeStruct((NW, L), jnp.float32),
    mesh=_mesh,
    scratch_types=[
        pltpu.VMEM((2, CH, B), jnp.float32),
        pltpu.VMEM((L,), jnp.float32),
        pltpu.SemaphoreType.DMA,
        pltpu.SemaphoreType.DMA,
    ],
    compiler_params=pltpu.CompilerParams(needs_layout_passes=False),
)


RTC = 128
LTC = 128
NCH_TC = B // LTC


def _tc_body(in_ref, out_ref):
    g = pl.program_id(0)
    rr = lax.broadcasted_iota(jnp.int32, (RTC, LTC), 0)
    cc = lax.broadcasted_iota(jnp.int32, (RTC, LTC), 1)

    dmask = cc == rr
    xg = in_ref[:, pl.ds(g * LTC, LTC)]
    d = jnp.max(jnp.where(dmask, xg, NEG), axis=1)
    in_ref[:, pl.ds(g * LTC, LTC)] = jnp.where(dmask, NEG, xg)

    neg = jnp.full((RTC, LTC), NEG, jnp.float32)
    t1, t2, t3 = neg, neg, neg
    for j in range(NCH_TC):
        x = in_ref[:, j * LTC:(j + 1) * LTC]
        n1 = jnp.maximum(t1, x)
        lo = jnp.minimum(t1, x)
        t2, lo2 = jnp.maximum(t2, lo), jnp.minimum(t2, lo)
        t3 = jnp.maximum(t3, lo2)
        t1 = n1

    partial = jnp.float32(0.0)
    for _ in range(K):
        m = jnp.max(t1, axis=1, keepdims=True)
        fl = jnp.min(jnp.where(t1 == m, cc, LTC), axis=1, keepdims=True)
        sel = cc == fl
        partial += jnp.sum(jnp.maximum(m[:, 0] - d + MARGIN, 0.0))
        t1 = jnp.where(sel, t2, t1)
        t2 = jnp.where(sel, t3, t2)
        t3 = jnp.where(sel, NEG, t3)

    @pl.when(g == 0)
    def _():
        out_ref[0, 0] = 0.0

    out_ref[0, 0] += partial


_tc_loss = pl.pallas_call(
    _tc_body,
    grid=(S_TC // RTC,),
    in_specs=[pl.BlockSpec((RTC, B), lambda g: (g, 0))],
    out_specs=pl.BlockSpec(memory_space=pltpu.SMEM),
    out_shape=jax.ShapeDtypeStruct((1, 1), jnp.float32),
)


def kernel(input, target):
    del target
    sc_part = _sc_loss(input)
    tc_part = _tc_loss(input)
    return (jnp.sum(sc_part) + tc_part[0, 0]) / (B * K)


# speedup vs baseline: 1.0176x; 1.0176x over previous
import functools

import jax
import jax.numpy as jnp
from jax import lax
from jax.experimental import pallas as pl
from jax.experimental.pallas import tpu as pltpu
from jax.experimental.pallas import tpu_sc as plsc

B = 4096
MARGIN = 0.8
K = 3
NEG = -3.0e38

S_TC = 2048


L = 16
NC, NS = 2, 16
NW = NC * NS
RPW = (B - S_TC) // NW
CH = 8
NCHUNK = RPW // CH
NVREG = B // L


def _tec_body(in_hbm, out_hbm, buf, stage, sem0, sem1):
    cid = lax.axis_index("c")
    sid = lax.axis_index("s")
    wid = sid * NC + cid
    base = S_TC + wid * RPW
    lane = lax.broadcasted_iota(jnp.int32, (L,), 0)
    neg = jnp.full((L,), NEG, jnp.float32)

    sems = (sem0, sem1)

    def start_chunk(k, slot):
        pltpu.async_copy(
            in_hbm.at[pl.ds(base + k * CH, CH)], buf.at[slot], sems[slot])

    def wait_chunk(k, slot):
        pltpu.make_async_copy(
            in_hbm.at[pl.ds(base + k * CH, CH)], buf.at[slot],
            sems[slot]).wait()

    def merge(t, x):
        t1, t2, t3 = t
        n1 = jnp.maximum(t1, x)
        lo = jnp.minimum(t1, x)
        n2 = jnp.maximum(t2, lo)
        lo2 = jnp.minimum(t2, lo)
        n3 = jnp.maximum(t3, lo2)
        return (n1, n2, n3)

    U = 8
    A = 4

    def process_row(i_global, slot, r_local, acc):
        c = i_global // L
        ldiag = i_global % L
        v = buf[slot, r_local, pl.ds(c * L, L)]
        dmask = lane == ldiag
        d = jnp.max(jnp.where(dmask, v, neg))
        buf[slot, r_local, pl.ds(c * L, L)] = jnp.where(dmask, neg, v)

        def scan_body(s, carry):
            ts = list(carry)
            for u in range(U):
                x = buf[slot, r_local, pl.ds((s * U + u) * L, L)]
                ts[u % A] = merge(ts[u % A], x)
            return tuple(ts)

        init = tuple((neg, neg, neg) for _ in range(A))
        ts = lax.fori_loop(0, NVREG // U, scan_body, init)
        t = ts[0]
        for a in range(1, A):
            o1, o2, o3 = ts[a]
            t = merge(merge(merge(t, o1), o2), o3)
        t1, t2, t3 = t

        for _ in range(K):
            m = jnp.max(t1)
            sel = lane == plsc.all_reduce_ffs(t1 == m)
            acc = acc + jnp.maximum(m - d + MARGIN, 0.0)
            t1 = jnp.where(sel, t2, t1)
            t2 = jnp.where(sel, t3, t2)
            t3 = jnp.where(sel, NEG, t3)
        return acc

    def chunk_pair(p, acc):
        start_chunk(2 * p + 1, 1)
        wait_chunk(2 * p, 0)

        def rows0(r, a):
            return process_row(base + (2 * p) * CH + r, 0, r, a)

        acc = lax.fori_loop(0, CH, rows0, acc)

        @pl.when(p + 1 < NCHUNK // 2)
        def _():
            start_chunk(2 * p + 2, 0)

        wait_chunk(2 * p + 1, 1)

        def rows1(r, a):
            return process_row(base + (2 * p + 1) * CH + r, 1, r, a)

        return lax.fori_loop(0, CH, rows1, acc)

    start_chunk(0, 0)
    acc = lax.fori_loop(0, NCHUNK // 2, chunk_pair, jnp.float32(0.0))

    stage[...] = jnp.where(lane == 0, acc, 0.0)
    pltpu.sync_copy(stage, out_hbm.at[wid])


_mesh = plsc.VectorSubcoreMesh(core_axis_name="c", subcore_axis_name="s")

_sc_loss = pl.kernel(
    _tec_body,
    out_type=jax.ShapeDtypeStruct((NW, L), jnp.float32),
    mesh=_mesh,
    scratch_types=[
        pltpu.VMEM((2, CH, B), jnp.float32),
        pltpu.VMEM((L,), jnp.float32),
        pltpu.SemaphoreType.DMA,
        pltpu.SemaphoreType.DMA,
    ],
    compiler_params=pltpu.CompilerParams(needs_layout_passes=False),
)


RTC = 128
LTC = 128
NCH_TC = B // LTC


def _tc_body(in_ref, out_ref):
    g = pl.program_id(0)
    rr = lax.broadcasted_iota(jnp.int32, (RTC, LTC), 0)
    cc = lax.broadcasted_iota(jnp.int32, (RTC, LTC), 1)

    dmask = cc == rr
    xg = in_ref[:, pl.ds(g * LTC, LTC)]
    d = jnp.max(jnp.where(dmask, xg, NEG), axis=1)
    in_ref[:, pl.ds(g * LTC, LTC)] = jnp.where(dmask, NEG, xg)

    neg = jnp.full((RTC, LTC), NEG, jnp.float32)
    t1, t2, t3 = neg, neg, neg
    for j in range(NCH_TC):
        x = in_ref[:, j * LTC:(j + 1) * LTC]
        n1 = jnp.maximum(t1, x)
        lo = jnp.minimum(t1, x)
        t2, lo2 = jnp.maximum(t2, lo), jnp.minimum(t2, lo)
        t3 = jnp.maximum(t3, lo2)
        t1 = n1

    partial = jnp.float32(0.0)
    for _ in range(K):
        m = jnp.max(t1, axis=1, keepdims=True)
        fl = jnp.min(jnp.where(t1 == m, cc, LTC), axis=1, keepdims=True)
        sel = cc == fl
        partial += jnp.sum(jnp.maximum(m[:, 0] - d + MARGIN, 0.0))
        t1 = jnp.where(sel, t2, t1)
        t2 = jnp.where(sel, t3, t2)
        t3 = jnp.where(sel, NEG, t3)

    @pl.when(g == 0)
    def _():
        out_ref[0, 0] = 0.0

    out_ref[0, 0] += partial


_tc_loss = pl.pallas_call(
    _tc_body,
    grid=(S_TC // RTC,),
    in_specs=[pl.BlockSpec((RTC, B), lambda g: (g, 0))],
    out_specs=pl.BlockSpec(memory_space=pltpu.SMEM),
    out_shape=jax.ShapeDtypeStruct((1, 1), jnp.float32),
)


def kernel(input, target):
    del target
    sc_part = _sc_loss(input)
    tc_part = _tc_loss(input)
    return (jnp.sum(sc_part) + tc_part[0, 0]) / (B * K)


# speedup vs baseline: 1.0560x; 1.0377x over previous
import functools

import jax
import jax.numpy as jnp
from jax import lax
from jax.experimental import pallas as pl
from jax.experimental.pallas import tpu as pltpu
from jax.experimental.pallas import tpu_sc as plsc

B = 4096
MARGIN = 0.8
K = 3
NEG = -3.0e38

S_TC = 2304


L = 16
NC, NS = 2, 16
NW = NC * NS
RPW = (B - S_TC) // NW
CH = 8
NCHUNK = RPW // CH
NVREG = B // L


def _tec_body(in_hbm, out_hbm, buf, stage, sem0, sem1):
    cid = lax.axis_index("c")
    sid = lax.axis_index("s")
    wid = sid * NC + cid
    base = S_TC + wid * RPW
    lane = lax.broadcasted_iota(jnp.int32, (L,), 0)
    neg = jnp.full((L,), NEG, jnp.float32)

    sems = (sem0, sem1)

    def start_chunk(k, slot):
        pltpu.async_copy(
            in_hbm.at[pl.ds(base + k * CH, CH)], buf.at[slot], sems[slot])

    def wait_chunk(k, slot):
        pltpu.make_async_copy(
            in_hbm.at[pl.ds(base + k * CH, CH)], buf.at[slot],
            sems[slot]).wait()

    def merge(t, x):
        t1, t2, t3 = t
        n1 = jnp.maximum(t1, x)
        lo = jnp.minimum(t1, x)
        n2 = jnp.maximum(t2, lo)
        lo2 = jnp.minimum(t2, lo)
        n3 = jnp.maximum(t3, lo2)
        return (n1, n2, n3)

    U = 16
    A = 4

    def process_row(i_global, slot, r_local, acc):
        c = i_global // L
        ldiag = i_global % L
        v = buf[slot, r_local, pl.ds(c * L, L)]
        dmask = lane == ldiag
        d = jnp.max(jnp.where(dmask, v, neg))
        buf[slot, r_local, pl.ds(c * L, L)] = jnp.where(dmask, neg, v)

        def scan_body(s, carry):
            ts = list(carry)
            for u in range(U):
                x = buf[slot, r_local, pl.ds((s * U + u) * L, L)]
                ts[u % A] = merge(ts[u % A], x)
            return tuple(ts)

        init = tuple((neg, neg, neg) for _ in range(A))
        ts = lax.fori_loop(0, NVREG // U, scan_body, init)
        t = ts[0]
        for a in range(1, A):
            o1, o2, o3 = ts[a]
            t = merge(merge(merge(t, o1), o2), o3)
        t1, t2, t3 = t

        for _ in range(K):
            m = jnp.max(t1)
            sel = lane == plsc.all_reduce_ffs(t1 == m)
            acc = acc + jnp.maximum(m - d + MARGIN, 0.0)
            t1 = jnp.where(sel, t2, t1)
            t2 = jnp.where(sel, t3, t2)
            t3 = jnp.where(sel, NEG, t3)
        return acc

    def chunk_pair(p, acc):
        start_chunk(2 * p + 1, 1)
        wait_chunk(2 * p, 0)

        def rows0(r, a):
            return process_row(base + (2 * p) * CH + r, 0, r, a)

        acc = lax.fori_loop(0, CH, rows0, acc)

        @pl.when(p + 1 < NCHUNK // 2)
        def _():
            start_chunk(2 * p + 2, 0)

        wait_chunk(2 * p + 1, 1)

        def rows1(r, a):
            return process_row(base + (2 * p + 1) * CH + r, 1, r, a)

        return lax.fori_loop(0, CH, rows1, acc)

    start_chunk(0, 0)
    acc = lax.fori_loop(0, NCHUNK // 2, chunk_pair, jnp.float32(0.0))

    stage[...] = jnp.where(lane == 0, acc, 0.0)
    pltpu.sync_copy(stage, out_hbm.at[wid])


_mesh = plsc.VectorSubcoreMesh(core_axis_name="c", subcore_axis_name="s")

_sc_loss = pl.kernel(
    _tec_body,
    out_type=jax.ShapeDtypeStruct((NW, L), jnp.float32),
    mesh=_mesh,
    scratch_types=[
        pltpu.VMEM((2, CH, B), jnp.float32),
        pltpu.VMEM((L,), jnp.float32),
        pltpu.SemaphoreType.DMA,
        pltpu.SemaphoreType.DMA,
    ],
    compiler_params=pltpu.CompilerParams(needs_layout_passes=False),
)


RTC = 128
LTC = 128
NCH_TC = B // LTC


def _tc_body(in_ref, out_ref):
    g = pl.program_id(0)
    rr = lax.broadcasted_iota(jnp.int32, (RTC, LTC), 0)
    cc = lax.broadcasted_iota(jnp.int32, (RTC, LTC), 1)

    dmask = cc == rr
    xg = in_ref[:, pl.ds(g * LTC, LTC)]
    d = jnp.max(jnp.where(dmask, xg, NEG), axis=1)
    in_ref[:, pl.ds(g * LTC, LTC)] = jnp.where(dmask, NEG, xg)

    neg = jnp.full((RTC, LTC), NEG, jnp.float32)
    t1, t2, t3 = neg, neg, neg
    for j in range(NCH_TC):
        x = in_ref[:, j * LTC:(j + 1) * LTC]
        n1 = jnp.maximum(t1, x)
        lo = jnp.minimum(t1, x)
        t2, lo2 = jnp.maximum(t2, lo), jnp.minimum(t2, lo)
        t3 = jnp.maximum(t3, lo2)
        t1 = n1

    partial = jnp.float32(0.0)
    for _ in range(K):
        m = jnp.max(t1, axis=1, keepdims=True)
        fl = jnp.min(jnp.where(t1 == m, cc, LTC), axis=1, keepdims=True)
        sel = cc == fl
        partial += jnp.sum(jnp.maximum(m[:, 0] - d + MARGIN, 0.0))
        t1 = jnp.where(sel, t2, t1)
        t2 = jnp.where(sel, t3, t2)
        t3 = jnp.where(sel, NEG, t3)

    @pl.when(g == 0)
    def _():
        out_ref[0, 0] = 0.0

    out_ref[0, 0] += partial


_tc_loss = pl.pallas_call(
    _tc_body,
    grid=(S_TC // RTC,),
    in_specs=[pl.BlockSpec((RTC, B), lambda g: (g, 0))],
    out_specs=pl.BlockSpec(memory_space=pltpu.SMEM),
    out_shape=jax.ShapeDtypeStruct((1, 1), jnp.float32),
)


def kernel(input, target):
    del target
    sc_part = _sc_loss(input)
    tc_part = _tc_loss(input)
    return (jnp.sum(sc_part) + tc_part[0, 0]) / (B * K)
